# Initial kernel scaffold; baseline (speedup 1.0000x reference)
import jax
import jax.numpy as jnp
from jax.experimental import pallas as pl


def kernel(x, W, b, dec_bias):
    raise NotImplementedError("write your pallas kernel here")


# speedup vs baseline: 5.9919x; 5.9919x over previous
import functools

import jax
import jax.numpy as jnp
from jax.experimental import pallas as pl
from jax.experimental.pallas import tpu as pltpu

INPUT_DIM = 2048
BOTTLENECK = 16384
K = 32


def _enc_body(w_ref, x_ref, b_ref, z1_ref):
    z = jax.lax.dot_general(
        x_ref[...], w_ref[...],
        dimension_numbers=(((1,), (1,)), ((), ())),
        preferred_element_type=jnp.float32,
    )
    z1_ref[...] = z + b_ref[...]


def _encoder(x, W, b2d, tok_blk, bn_blk):
    n_tok = x.shape[0]
    grid = (BOTTLENECK // bn_blk, n_tok // tok_blk)
    return pl.pallas_call(
        _enc_body,
        grid=grid,
        in_specs=[
            pl.BlockSpec((bn_blk, INPUT_DIM), lambda j, i: (j, 0)),
            pl.BlockSpec((tok_blk, INPUT_DIM), lambda j, i: (i, 0)),
            pl.BlockSpec((1, bn_blk), lambda j, i: (0, j)),
        ],
        out_specs=pl.BlockSpec((tok_blk, bn_blk), lambda j, i: (i, j)),
        out_shape=jax.ShapeDtypeStruct((n_tok, BOTTLENECK), jnp.float32),
        compiler_params=pltpu.CompilerParams(
            dimension_semantics=("arbitrary", "arbitrary"),
        ),
    )(W, x, b2d)


def _topk_body(z1_ref, a1_ref, u_ref):
    u_ref[...] = z1_ref[...]

    def body(_, thr):
        m = jnp.max(u_ref[...], axis=1, keepdims=True)
        u_ref[...] = jnp.where(u_ref[...] >= m, -jnp.inf, u_ref[...])
        return m

    thr = jax.lax.fori_loop(
        0, K, body, jnp.zeros((u_ref.shape[0], 1), jnp.float32))
    z = z1_ref[...]
    a1_ref[...] = jnp.where(z >= thr, z, 0.0)


def _topk_mask(z1, tok_blk):
    n_tok = z1.shape[0]
    return pl.pallas_call(
        _topk_body,
        grid=(n_tok // tok_blk,),
        in_specs=[pl.BlockSpec((tok_blk, BOTTLENECK), lambda i: (i, 0))],
        out_specs=pl.BlockSpec((tok_blk, BOTTLENECK), lambda i: (i, 0)),
        out_shape=jax.ShapeDtypeStruct((n_tok, BOTTLENECK), jnp.float32),
        scratch_shapes=[pltpu.VMEM((tok_blk, BOTTLENECK), jnp.float32)],
        compiler_params=pltpu.CompilerParams(
            dimension_semantics=("arbitrary",),
        ),
    )(z1)


def _dec_body(a1_ref, w_ref, db_ref, z2_ref, acc_ref, *, n_kc):
    kc = pl.program_id(1)

    @pl.when(kc == 0)
    def _():
        acc_ref[...] = jnp.zeros_like(acc_ref)

    acc_ref[...] += jax.lax.dot_general(
        a1_ref[...], w_ref[...],
        dimension_numbers=(((1,), (0,)), ((), ())),
        preferred_element_type=jnp.float32,
    )

    @pl.when(kc == n_kc - 1)
    def _():
        z2_ref[...] = acc_ref[...] + db_ref[...]


def _decoder(a1, W, db2d, tok_blk, kc_blk):
    n_tok = a1.shape[0]
    n_kc = BOTTLENECK // kc_blk
    grid = (n_tok // tok_blk, n_kc)
    return pl.pallas_call(
        functools.partial(_dec_body, n_kc=n_kc),
        grid=grid,
        in_specs=[
            pl.BlockSpec((tok_blk, kc_blk), lambda i, k: (i, k)),
            pl.BlockSpec((kc_blk, INPUT_DIM), lambda i, k: (k, 0)),
            pl.BlockSpec((1, INPUT_DIM), lambda i, k: (0, 0)),
        ],
        out_specs=pl.BlockSpec((tok_blk, INPUT_DIM), lambda i, k: (i, 0)),
        out_shape=jax.ShapeDtypeStruct((n_tok, INPUT_DIM), jnp.float32),
        scratch_shapes=[pltpu.VMEM((tok_blk, INPUT_DIM), jnp.float32)],
        compiler_params=pltpu.CompilerParams(
            dimension_semantics=("arbitrary", "arbitrary"),
        ),
    )(a1, W, db2d)


def kernel(x, W, b, dec_bias):
    if x.ndim == 1:
        x = x[None, :]
    n_tok = x.shape[0]
    b2d = b.reshape(1, BOTTLENECK)
    db2d = dec_bias.reshape(1, INPUT_DIM)

    tok_blk_mm = min(512, n_tok)
    z1 = _encoder(x, W, b2d, tok_blk_mm, 1024)
    a1 = _topk_mask(z1, min(128, n_tok))
    z2 = _decoder(a1, W, db2d, tok_blk_mm, 2048)
    return z2
